# baseline (speedup 1.0000x reference)
import functools

import jax
import jax.numpy as jnp
from jax import lax
from jax.experimental import pallas as pl
from jax.experimental.pallas import tpu as pltpu
from jax.experimental.pallas import tpu_sc as plsc

_NUM_CORES = 2
_NUM_SUBCORES = 16
_NUM_WORKERS = _NUM_CORES * _NUM_SUBCORES

_LANES = 16
_K = 4
_UNIT_B = _K * 128


@functools.partial(jax.jit, static_argnums=(2, 3, 4))
def _sc_embed(idx_t, table, batch, hist, dim):
    n_btg = batch // _UNIT_B
    n_units = hist * n_btg
    upw = n_units // _NUM_WORKERS
    assert n_units % _NUM_WORKERS == 0 and upw % 2 == 0
    e_hi = dim // 8

    mesh = plsc.VectorSubcoreMesh(core_axis_name="c", subcore_axis_name="s")

    @functools.partial(
        pl.kernel,
        out_type=jax.ShapeDtypeStruct(
            (hist, e_hi, batch // 128, 8, 128), jnp.float32
        ),
        mesh=mesh,
        scratch_types=[
            pltpu.VMEM((_UNIT_B,), jnp.int32),
            pltpu.VMEM((_UNIT_B,), jnp.int32),
            pltpu.VMEM((_UNIT_B, dim), jnp.float32),
            pltpu.VMEM((_UNIT_B, dim), jnp.float32),
            pltpu.VMEM((e_hi, _K, 8, 128), jnp.float32),
            pltpu.VMEM((e_hi, _K, 8, 128), jnp.float32),
            pltpu.SemaphoreType.DMA,
            pltpu.SemaphoreType.DMA,
            pltpu.SemaphoreType.DMA,
            pltpu.SemaphoreType.DMA,
            pltpu.SemaphoreType.DMA,
            pltpu.SemaphoreType.DMA,
        ],
        compiler_params=pltpu.CompilerParams(
            use_tc_tiling_on_sc=False, needs_layout_passes=False
        ),
    )
    def k(idx_hbm, table_hbm, out_hbm,
          ib0, ib1, rb0, rb1, tb0, tb1, is0, is1, gs0, gs1, os0, os1):
        wid = lax.axis_index("s") * _NUM_CORES + lax.axis_index("c")
        u0 = wid * upw
        ibufs = (ib0, ib1)
        rbufs = (rb0, rb1)
        tbufs = (tb0, tb1)
        isems = (is0, is1)
        gsems = (gs0, gs1)
        osems = (os0, os1)
        iota = lax.iota(jnp.int32, _LANES)

        def idx_off(u):
            return (u // n_btg) * batch + (u % n_btg) * _UNIT_B

        def out_dst(u):
            return out_hbm.at[u // n_btg, :, pl.ds((u % n_btg) * _K, _K)]

        pltpu.sync_copy(idx_hbm.at[pl.ds(idx_off(u0), _UNIT_B)], ib0)
        pltpu.async_copy(table_hbm.at[ib0], rb0, gs0)
        pltpu.async_copy(idx_hbm.at[pl.ds(idx_off(u0 + 1), _UNIT_B)], ib1, is1)

        def pair(p, carry):
            for ph in range(2):
                cur = ph
                nxt = 1 - ph
                i = p * 2 + ph
                u = u0 + i

                pltpu.make_async_copy(
                    table_hbm.at[ibufs[cur]], rbufs[cur], gsems[cur]
                ).wait()

                @pl.when(i + 2 < upw)
                def _():
                    pltpu.async_copy(
                        idx_hbm.at[pl.ds(idx_off(u + 2), _UNIT_B)],
                        ibufs[cur],
                        isems[cur],
                    )

                @pl.when(i >= 2)
                def _():
                    pltpu.make_async_copy(
                        tbufs[cur], out_dst(u), osems[cur]
                    ).wait()

                def bg_step(bg, c):
                    for btl in range(_K):
                        rvec = iota + (btl * 128 + bg * _LANES)
                        for e in range(dim):
                            col = jnp.full((_LANES,), e, jnp.int32)
                            v = plsc.load_gather(rbufs[cur], [rvec, col])
                            tbufs[cur][
                                e // 8, btl, e % 8, pl.ds(bg * _LANES, _LANES)
                            ] = v
                    return c

                lax.fori_loop(0, 128 // _LANES, bg_step, 0)

                pltpu.async_copy(tbufs[cur], out_dst(u), osems[cur])

                @pl.when(i + 1 < upw)
                def _():
                    pltpu.make_async_copy(
                        idx_hbm.at[pl.ds(idx_off(u + 1), _UNIT_B)],
                        ibufs[nxt],
                        isems[nxt],
                    ).wait()
                    pltpu.async_copy(table_hbm.at[ibufs[nxt]], rbufs[nxt], gsems[nxt])

            return carry

        lax.fori_loop(0, upw // 2, pair, 0)

        for cur in range(2):
            pltpu.make_async_copy(
                tbufs[cur], out_dst(u0 + upw - 2 + cur), osems[cur]
            ).wait()

    return k(idx_t, table)


def kernel(inputs, embeddings):
    batch, hist = inputs.shape
    _, dim = embeddings.shape
    idx_t = inputs.T.reshape(batch * hist).astype(jnp.int32)
    out5 = _sc_embed(idx_t, embeddings, batch, hist, dim)
    return out5.transpose(2, 4, 0, 1, 3).reshape(batch, hist, dim)


# speedup vs baseline: 1.3498x; 1.3498x over previous
import functools

import jax
import jax.numpy as jnp
from jax import lax
from jax.experimental import pallas as pl
from jax.experimental.pallas import tpu as pltpu
from jax.experimental.pallas import tpu_sc as plsc

_NUM_CORES = 2
_NUM_SUBCORES = 16
_NUM_WORKERS = _NUM_CORES * _NUM_SUBCORES

_LANES = 16
_K = 4
_UNIT_B = _K * 128


@functools.partial(jax.jit, static_argnums=(2, 3, 4))
def _sc_embed(idx_t, table, batch, hist, dim):
    n_btg = batch // _UNIT_B
    n_units = hist * n_btg
    upw = n_units // _NUM_WORKERS
    assert n_units % _NUM_WORKERS == 0 and upw % 2 == 0
    e_hi = dim // 8

    mesh = plsc.VectorSubcoreMesh(core_axis_name="c", subcore_axis_name="s")

    @functools.partial(
        pl.kernel,
        out_type=jax.ShapeDtypeStruct(
            (hist, e_hi, batch // 128, 8, 128), jnp.float32
        ),
        mesh=mesh,
        scratch_types=[
            pltpu.VMEM((_UNIT_B,), jnp.int32),
            pltpu.VMEM((_UNIT_B,), jnp.int32),
            pltpu.VMEM((_UNIT_B, dim), jnp.float32),
            pltpu.VMEM((_UNIT_B, dim), jnp.float32),
            pltpu.VMEM((e_hi, _K, 8, 128), jnp.float32),
            pltpu.VMEM((e_hi, _K, 8, 128), jnp.float32),
            pltpu.SemaphoreType.DMA,
            pltpu.SemaphoreType.DMA,
            pltpu.SemaphoreType.DMA,
            pltpu.SemaphoreType.DMA,
            pltpu.SemaphoreType.DMA,
            pltpu.SemaphoreType.DMA,
        ],
        compiler_params=pltpu.CompilerParams(
            use_tc_tiling_on_sc=False, needs_layout_passes=False
        ),
    )
    def k(idx_hbm, table_hbm, out_hbm,
          ib0, ib1, rb0, rb1, tb0, tb1, is0, is1, gs0, gs1, os0, os1):
        wid = lax.axis_index("s") * _NUM_CORES + lax.axis_index("c")
        u0 = wid * upw
        ibufs = (ib0, ib1)
        rbufs = (rb0, rb1)
        tbufs = (tb0, tb1)
        isems = (is0, is1)
        gsems = (gs0, gs1)
        osems = (os0, os1)
        iota = lax.iota(jnp.int32, _LANES)

        def idx_off(u):
            return (u // n_btg) * batch + (u % n_btg) * _UNIT_B

        def out_dst(u):
            return out_hbm.at[u // n_btg, :, pl.ds((u % n_btg) * _K, _K)]

        pltpu.sync_copy(idx_hbm.at[pl.ds(idx_off(u0), _UNIT_B)], ib0)
        pltpu.async_copy(table_hbm.at[ib0], rb0, gs0)
        pltpu.async_copy(idx_hbm.at[pl.ds(idx_off(u0 + 1), _UNIT_B)], ib1, is1)

        def pair(p, carry):
            for ph in range(2):
                cur = ph
                nxt = 1 - ph
                i = p * 2 + ph
                u = u0 + i

                pltpu.make_async_copy(
                    table_hbm.at[ibufs[cur]], rbufs[cur], gsems[cur]
                ).wait()

                @pl.when(i + 2 < upw)
                def _():
                    pltpu.async_copy(
                        idx_hbm.at[pl.ds(idx_off(u + 2), _UNIT_B)],
                        ibufs[cur],
                        isems[cur],
                    )

                @pl.when(i >= 2)
                def _():
                    pltpu.make_async_copy(
                        tbufs[cur], out_dst(u), osems[cur]
                    ).wait()

                @plsc.parallel_loop(0, (128 // _LANES) * _K, unroll=4)
                def _(j):
                    bg = j // _K
                    btl = j % _K
                    rvec = iota + (btl * 128 + bg * _LANES)
                    for e in range(dim):
                        col = jnp.full((_LANES,), e, jnp.int32)
                        v = plsc.load_gather(rbufs[cur], [rvec, col])
                        tbufs[cur][
                            e // 8, btl, e % 8, pl.ds(bg * _LANES, _LANES)
                        ] = v

                pltpu.async_copy(tbufs[cur], out_dst(u), osems[cur])

                @pl.when(i + 1 < upw)
                def _():
                    pltpu.make_async_copy(
                        idx_hbm.at[pl.ds(idx_off(u + 1), _UNIT_B)],
                        ibufs[nxt],
                        isems[nxt],
                    ).wait()
                    pltpu.async_copy(table_hbm.at[ibufs[nxt]], rbufs[nxt], gsems[nxt])

            return carry

        lax.fori_loop(0, upw // 2, pair, 0)

        for cur in range(2):
            pltpu.make_async_copy(
                tbufs[cur], out_dst(u0 + upw - 2 + cur), osems[cur]
            ).wait()

    return k(idx_t, table)


def kernel(inputs, embeddings):
    batch, hist = inputs.shape
    _, dim = embeddings.shape
    idx_t = inputs.T.reshape(batch * hist).astype(jnp.int32)
    out5 = _sc_embed(idx_t, embeddings, batch, hist, dim)
    return out5.transpose(2, 4, 0, 1, 3).reshape(batch, hist, dim)


# speedup vs baseline: 1.5455x; 1.1450x over previous
import functools

import jax
import jax.numpy as jnp
from jax import lax
from jax.experimental import pallas as pl
from jax.experimental.pallas import tpu as pltpu
from jax.experimental.pallas import tpu_sc as plsc

_NUM_CORES = 2
_NUM_SUBCORES = 16
_NUM_WORKERS = _NUM_CORES * _NUM_SUBCORES

_LANES = 16
_K = 4
_UNIT_B = _K * 128


@functools.partial(jax.jit, static_argnums=(2, 3, 4))
def _sc_embed(idx_t, table, batch, hist, dim):
    n_btg = batch // _UNIT_B
    n_units = hist * n_btg
    upw = n_units // _NUM_WORKERS
    assert n_units % _NUM_WORKERS == 0 and upw % 2 == 0
    e_hi = dim // 8

    mesh = plsc.VectorSubcoreMesh(core_axis_name="c", subcore_axis_name="s")

    @functools.partial(
        pl.kernel,
        out_type=jax.ShapeDtypeStruct(
            (hist, e_hi, batch // 128, 8, 128), jnp.float32
        ),
        mesh=mesh,
        scratch_types=[
            pltpu.VMEM((_UNIT_B,), jnp.int32),
            pltpu.VMEM((_UNIT_B,), jnp.int32),
            pltpu.VMEM((_UNIT_B, dim), jnp.float32),
            pltpu.VMEM((_UNIT_B, dim), jnp.float32),
            pltpu.VMEM((e_hi, _K, 8, 128), jnp.float32),
            pltpu.VMEM((e_hi, _K, 8, 128), jnp.float32),
            pltpu.SemaphoreType.DMA,
            pltpu.SemaphoreType.DMA,
            pltpu.SemaphoreType.DMA,
            pltpu.SemaphoreType.DMA,
            pltpu.SemaphoreType.DMA,
            pltpu.SemaphoreType.DMA,
        ],
        compiler_params=pltpu.CompilerParams(
            use_tc_tiling_on_sc=False, needs_layout_passes=False
        ),
    )
    def k(idx_hbm, table_hbm, out_hbm,
          ib0, ib1, rb0, rb1, tb0, tb1, is0, is1, gs0, gs1, os0, os1):
        wid = lax.axis_index("s") * _NUM_CORES + lax.axis_index("c")
        u0 = wid * upw
        ibufs = (ib0, ib1)
        rbufs = (rb0, rb1)
        tbufs = (tb0, tb1)
        isems = (is0, is1)
        gsems = (gs0, gs1)
        osems = (os0, os1)
        iota = lax.iota(jnp.int32, _LANES)

        def idx_off(u):
            return (u // n_btg) * batch + (u % n_btg) * _UNIT_B

        def out_dst(u):
            return out_hbm.at[u // n_btg, :, pl.ds((u % n_btg) * _K, _K)]

        pltpu.sync_copy(idx_hbm.at[pl.ds(idx_off(u0), _UNIT_B)], ib0)
        pltpu.async_copy(table_hbm.at[ib0], rb0, gs0)
        pltpu.async_copy(idx_hbm.at[pl.ds(idx_off(u0 + 1), _UNIT_B)], ib1, is1)

        def pair(p, carry):
            for ph in range(2):
                cur = ph
                nxt = 1 - ph
                i = p * 2 + ph
                u = u0 + i

                pltpu.make_async_copy(
                    table_hbm.at[ibufs[cur]], rbufs[cur], gsems[cur]
                ).wait()

                @pl.when(i + 1 < upw)
                def _():
                    pltpu.make_async_copy(
                        idx_hbm.at[pl.ds(idx_off(u + 1), _UNIT_B)],
                        ibufs[nxt],
                        isems[nxt],
                    ).wait()
                    pltpu.async_copy(table_hbm.at[ibufs[nxt]], rbufs[nxt], gsems[nxt])

                @pl.when(i + 2 < upw)
                def _():
                    pltpu.async_copy(
                        idx_hbm.at[pl.ds(idx_off(u + 2), _UNIT_B)],
                        ibufs[cur],
                        isems[cur],
                    )

                @pl.when(i >= 2)
                def _():
                    pltpu.make_async_copy(
                        tbufs[cur], out_dst(u), osems[cur]
                    ).wait()

                @plsc.parallel_loop(0, (128 // _LANES) * _K, unroll=8)
                def _(j):
                    bg = j // _K
                    btl = j % _K
                    rvec = iota + (btl * 128 + bg * _LANES)
                    for e in range(dim):
                        col = jnp.full((_LANES,), e, jnp.int32)
                        v = plsc.load_gather(rbufs[cur], [rvec, col])
                        tbufs[cur][
                            e // 8, btl, e % 8, pl.ds(bg * _LANES, _LANES)
                        ] = v

                pltpu.async_copy(tbufs[cur], out_dst(u), osems[cur])

            return carry

        lax.fori_loop(0, upw // 2, pair, 0)

        for cur in range(2):
            pltpu.make_async_copy(
                tbufs[cur], out_dst(u0 + upw - 2 + cur), osems[cur]
            ).wait()

    return k(idx_t, table)


def kernel(inputs, embeddings):
    batch, hist = inputs.shape
    _, dim = embeddings.shape
    idx_t = inputs.T.reshape(batch * hist).astype(jnp.int32)
    out5 = _sc_embed(idx_t, embeddings, batch, hist, dim)
    return out5.transpose(2, 4, 0, 1, 3).reshape(batch, hist, dim)


# speedup vs baseline: 6.9149x; 4.4741x over previous
import functools

import jax
import jax.numpy as jnp
from jax import lax
from jax.experimental import pallas as pl
from jax.experimental.pallas import tpu as pltpu
from jax.experimental.pallas import tpu_sc as plsc

_NUM_CORES = 2
_NUM_SUBCORES = 16
_NUM_WORKERS = _NUM_CORES * _NUM_SUBCORES

_LANES = 16
_K = 4
_UNIT_B = _K * 128


@functools.partial(jax.jit, static_argnums=(2, 3, 4))
def _sc_embed(idx_t, table, batch, hist, dim):
    n_btg = batch // _UNIT_B
    n_units = hist * n_btg
    upw = n_units // _NUM_WORKERS
    assert n_units % _NUM_WORKERS == 0 and upw % 2 == 0
    e_hi = dim // 8

    mesh = plsc.VectorSubcoreMesh(core_axis_name="c", subcore_axis_name="s")

    @functools.partial(
        pl.kernel,
        out_type=jax.ShapeDtypeStruct(
            (hist, e_hi, batch // 128, 8, 128), jnp.float32
        ),
        mesh=mesh,
        scratch_types=[
            pltpu.VMEM((_UNIT_B,), jnp.int32),
            pltpu.VMEM((_UNIT_B,), jnp.int32),
            pltpu.VMEM((_UNIT_B, dim), jnp.float32),
            pltpu.VMEM((_UNIT_B, dim), jnp.float32),
            pltpu.VMEM((_K * dim, 129), jnp.float32),
            pltpu.VMEM((_K * dim, 129), jnp.float32),
            pltpu.SemaphoreType.DMA,
            pltpu.SemaphoreType.DMA,
            pltpu.SemaphoreType.DMA,
            pltpu.SemaphoreType.DMA,
            pltpu.SemaphoreType.DMA,
            pltpu.SemaphoreType.DMA,
        ],
        compiler_params=pltpu.CompilerParams(
            use_tc_tiling_on_sc=False, needs_layout_passes=False
        ),
    )
    def k(idx_hbm, table_hbm, out_hbm,
          ib0, ib1, rb0, rb1, tb0, tb1, is0, is1, gs0, gs1, os0, os1):
        wid = lax.axis_index("s") * _NUM_CORES + lax.axis_index("c")
        u0 = wid * upw
        ibufs = (ib0, ib1)
        rbufs = (rb0, rb1)
        tbufs = (tb0, tb1)
        isems = (is0, is1)
        gsems = (gs0, gs1)
        osems = (os0, os1)
        iota = lax.iota(jnp.int32, _LANES)

        def idx_off(u):
            return (u // n_btg) * batch + (u % n_btg) * _UNIT_B

        def unit_stores(u, tb, sem):
            h = u // n_btg
            btg = u % n_btg
            for ei in range(e_hi):
                for btl in range(_K):
                    src = tb.at[pl.ds(btl * dim + ei * 8, 8), pl.ds(0, 128)]
                    dst = out_hbm.at[h, ei, btg * _K + btl]
                    yield pltpu.make_async_copy(src, dst, sem)

        pltpu.sync_copy(idx_hbm.at[pl.ds(idx_off(u0), _UNIT_B)], ib0)
        pltpu.async_copy(table_hbm.at[ib0], rb0, gs0)
        pltpu.async_copy(idx_hbm.at[pl.ds(idx_off(u0 + 1), _UNIT_B)], ib1, is1)

        def pair(p, carry):
            for ph in range(2):
                cur = ph
                nxt = 1 - ph
                i = p * 2 + ph
                u = u0 + i

                pltpu.make_async_copy(
                    table_hbm.at[ibufs[cur]], rbufs[cur], gsems[cur]
                ).wait()

                @pl.when(i + 1 < upw)
                def _():
                    pltpu.make_async_copy(
                        idx_hbm.at[pl.ds(idx_off(u + 1), _UNIT_B)],
                        ibufs[nxt],
                        isems[nxt],
                    ).wait()
                    pltpu.async_copy(table_hbm.at[ibufs[nxt]], rbufs[nxt], gsems[nxt])

                @pl.when(i + 2 < upw)
                def _():
                    pltpu.async_copy(
                        idx_hbm.at[pl.ds(idx_off(u + 2), _UNIT_B)],
                        ibufs[cur],
                        isems[cur],
                    )

                @pl.when(i >= 2)
                def _():
                    for cp in unit_stores(u, tbufs[cur], osems[cur]):
                        cp.wait()

                @plsc.parallel_loop(0, _UNIT_B, unroll=8)
                def _(r):
                    btl = r // 128
                    bl = r - btl * 128
                    colv = jnp.zeros((_LANES,), jnp.int32) + bl
                    for k2 in range(dim // _LANES):
                        v = rbufs[cur][r, pl.ds(k2 * _LANES, _LANES)]
                        rowv = iota + (k2 * _LANES + btl * dim)
                        plsc.store_scatter(tbufs[cur], [rowv, colv], v)

                for cp in unit_stores(u, tbufs[cur], osems[cur]):
                    cp.start()

            return carry

        lax.fori_loop(0, upw // 2, pair, 0)

        for cur in range(2):
            for cp in unit_stores(u0 + upw - 2 + cur, tbufs[cur], osems[cur]):
                cp.wait()

    return k(idx_t, table)


def kernel(inputs, embeddings):
    batch, hist = inputs.shape
    _, dim = embeddings.shape
    idx_t = inputs.T.reshape(batch * hist).astype(jnp.int32)
    out5 = _sc_embed(idx_t, embeddings, batch, hist, dim)
    return out5.transpose(2, 4, 0, 1, 3).reshape(batch, hist, dim)


# speedup vs baseline: 6.9677x; 1.0076x over previous
import functools

import jax
import jax.numpy as jnp
from jax import lax
from jax.experimental import pallas as pl
from jax.experimental.pallas import tpu as pltpu
from jax.experimental.pallas import tpu_sc as plsc

_NUM_CORES = 2
_NUM_SUBCORES = 16
_NUM_WORKERS = _NUM_CORES * _NUM_SUBCORES

_LANES = 16
_K = 4
_UNIT_B = _K * 128


@functools.partial(jax.jit, static_argnums=(2, 3, 4))
def _sc_embed(idx_t, table, batch, hist, dim):
    n_btg = batch // _UNIT_B
    n_units = hist * n_btg
    upw = n_units // _NUM_WORKERS
    assert n_units % _NUM_WORKERS == 0 and upw % 2 == 0
    e_hi = dim // 8

    mesh = plsc.VectorSubcoreMesh(core_axis_name="c", subcore_axis_name="s")

    @functools.partial(
        pl.kernel,
        out_type=jax.ShapeDtypeStruct(
            (hist, e_hi, batch // 128, 8, 128), jnp.float32
        ),
        mesh=mesh,
        scratch_types=[
            pltpu.VMEM((_K, 128), jnp.int32),
            pltpu.VMEM((_K, 128), jnp.int32),
            pltpu.VMEM((_K, 128, dim), jnp.float32),
            pltpu.VMEM((_K, 128, dim), jnp.float32),
            pltpu.VMEM((_K * dim, 129), jnp.float32),
            pltpu.VMEM((_K * dim, 129), jnp.float32),
            pltpu.SemaphoreType.DMA,
            pltpu.SemaphoreType.DMA,
            pltpu.SemaphoreType.DMA,
            pltpu.SemaphoreType.DMA,
            pltpu.SemaphoreType.DMA,
            pltpu.SemaphoreType.DMA,
        ],
        compiler_params=pltpu.CompilerParams(
            use_tc_tiling_on_sc=False, needs_layout_passes=False
        ),
    )
    def k(idx_hbm, table_hbm, out_hbm,
          ib0, ib1, rb0, rb1, tb0, tb1, is0, is1, gs0, gs1, os0, os1):
        wid = lax.axis_index("s") * _NUM_CORES + lax.axis_index("c")
        u0 = wid * upw
        ibufs = (ib0, ib1)
        rbufs = (rb0, rb1)
        tbufs = (tb0, tb1)
        isems = (is0, is1)
        gsems = (gs0, gs1)
        osems = (os0, os1)
        iota = lax.iota(jnp.int32, _LANES)

        def idx_src(u):
            h = u // n_btg
            btg = u % n_btg
            return idx_hbm.at[h // 8, pl.ds(btg * _K, _K), h % 8]

        def unit_stores(u, tb, sem):
            h = u // n_btg
            btg = u % n_btg
            for ei in range(e_hi):
                for btl in range(_K):
                    src = tb.at[pl.ds(btl * dim + ei * 8, 8), pl.ds(0, 128)]
                    dst = out_hbm.at[h, ei, btg * _K + btl]
                    yield pltpu.make_async_copy(src, dst, sem)

        def gather_unit(ib, rb, sem):
            for q in range(_K):
                yield pltpu.make_async_copy(table_hbm.at[ib.at[q]], rb.at[q], sem)

        pltpu.sync_copy(idx_src(u0), ib0)
        for cp in gather_unit(ib0, rb0, gs0):
            cp.start()
        pltpu.async_copy(idx_src(u0 + 1), ib1, is1)

        def pair(p, carry):
            for ph in range(2):
                cur = ph
                nxt = 1 - ph
                i = p * 2 + ph
                u = u0 + i

                for cp in gather_unit(ibufs[cur], rbufs[cur], gsems[cur]):
                    cp.wait()

                @pl.when(i + 1 < upw)
                def _():
                    pltpu.make_async_copy(
                        idx_src(u + 1), ibufs[nxt], isems[nxt]
                    ).wait()
                    for cp in gather_unit(ibufs[nxt], rbufs[nxt], gsems[nxt]):
                        cp.start()

                @pl.when(i + 2 < upw)
                def _():
                    pltpu.async_copy(idx_src(u + 2), ibufs[cur], isems[cur])

                @pl.when(i >= 2)
                def _():
                    for cp in unit_stores(u, tbufs[cur], osems[cur]):
                        cp.wait()

                @plsc.parallel_loop(0, _UNIT_B, unroll=8)
                def _(r):
                    btl = r // 128
                    bl = r - btl * 128
                    colv = jnp.zeros((_LANES,), jnp.int32) + bl
                    for k2 in range(dim // _LANES):
                        v = rbufs[cur][btl, bl, pl.ds(k2 * _LANES, _LANES)]
                        rowv = iota + (k2 * _LANES + btl * dim)
                        plsc.store_scatter(tbufs[cur], [rowv, colv], v)

                for cp in unit_stores(u, tbufs[cur], osems[cur]):
                    cp.start()

            return carry

        lax.fori_loop(0, upw // 2, pair, 0)

        for cur in range(2):
            for cp in unit_stores(u0 + upw - 2 + cur, tbufs[cur], osems[cur]):
                cp.wait()

    return k(idx_t, table)


def kernel(inputs, embeddings):
    batch, hist = inputs.shape
    _, dim = embeddings.shape
    idx4 = (
        inputs.astype(jnp.int32)
        .reshape(batch // 128, 128, hist // 8, 8)
        .transpose(2, 0, 3, 1)
    )
    out5 = _sc_embed(idx4, embeddings, batch, hist, dim)
    return out5.transpose(2, 4, 0, 1, 3).reshape(batch, hist, dim)
